# baseline (speedup 1.0000x reference)
import functools

import jax
import jax.numpy as jnp
from jax import lax
from jax.experimental import pallas as pl
from jax.experimental.pallas import tpu as pltpu
from jax.experimental.pallas import tpu_sc as plsc

USER_N = 5000
ITEM_N = 5000
N = USER_N + ITEM_N
NNZ = 320000
EMB = 128
NLAYERS = 3

NC = 2
NS = 16
NW = NC * NS
EPT = NNZ // NW
CHUNK = 48
NCH = 208
TAIL = EPT - NCH * CHUNK
NGRP = CHUNK // 16

STRIPE = 624
TAIL0 = N - NS * STRIPE
TAIL_OFF = NS * STRIPE

_mesh = plsc.VectorSubcoreMesh(
    core_axis_name="c", subcore_axis_name="s", num_cores=NC, num_subcores=NS
)

_DNUMS = lax.GatherDimensionNumbers(
    offset_dims=(), collapsed_slice_dims=(0,), start_index_map=(0,))


def _splat(vals16, lane):
    return lax.gather(vals16, jnp.full((16, 1), lane, jnp.int32), _DNUMS,
                      slice_sizes=(1,),
                      mode=lax.GatherScatterMode.PROMISE_IN_BOUNDS)


def _sc_layer_body(ego, pk, pv, pkt, pvt, zeros, part0, part1,
                   ib_a, ib_b, ibt, vb_a, vb_b, vbt, rb_a, rb_b, rbt,
                   gb_a, gb_b, sb_a, sb_b,
                   acc, sem_ga, sem_gb, sem_sa, sem_sb):
    c = lax.axis_index("c")
    s = lax.axis_index("s")
    wid = c * NS + s

    row0 = s * STRIPE
    pltpu.sync_copy(zeros.at[pl.ds(row0, STRIPE)], acc.at[pl.ds(row0, STRIPE)])

    @pl.when(s == 0)
    def _():
        pltpu.sync_copy(zeros.at[pl.ds(TAIL_OFF, TAIL0)],
                        acc.at[pl.ds(TAIL_OFF, TAIL0)])

    plsc.subcore_barrier()

    pltpu.sync_copy(pkt.at[wid], ibt)
    pltpu.sync_copy(pvt.at[wid], vbt)
    pltpu.async_copy(ego.at[ibt.at[0]], gb_a.at[pl.ds(0, 16)], sem_ga).wait()
    rbt[pl.ds(0, 16)] = ibt[1, pl.ds(0, 16)]
    vals16 = vbt[0, pl.ds(0, 16)]
    for lane in range(16):
        v = _splat(vals16, lane)
        for k in range(EMB // 16):
            sb_a[lane, pl.ds(k * 16, 16)] = gb_a[lane, pl.ds(k * 16, 16)] * v
    pltpu.sync_copy(sb_a.at[pl.ds(0, 16)], acc.at[rbt], add=True)

    def scale(j, ib, vb, rb, gb, sb):
        def group(g, carry):
            rb[pl.ds(g * 16, 16)] = ib[1, pl.ds(g * 16, 16)]
            vals = vb[0, pl.ds(g * 16, 16)]
            for lane in range(16):
                v = _splat(vals, lane)
                e = g * 16 + lane
                for k in range(EMB // 16):
                    sb[e, pl.ds(k * 16, 16)] = gb[e, pl.ds(k * 16, 16)] * v
            return carry
        lax.fori_loop(0, NGRP, group, 0)

    def segment(j, ib, vb, rb, gb, sb, sem_g, sem_s, drain, prefetch):
        pltpu.make_async_copy(ego.at[ib.at[0]], gb, sem_g).wait()
        if drain:
            pltpu.make_async_copy(sb, acc.at[rb], sem_s).wait()
        scale(j, ib, vb, rb, gb, sb)
        pltpu.async_copy(sb, acc.at[rb], sem_s, add=True)
        if prefetch:
            pltpu.sync_copy(pk.at[wid, j + 2], ib)
            pltpu.sync_copy(pv.at[wid, j + 2], vb)
            pltpu.async_copy(ego.at[ib.at[0]], gb, sem_g)

    pltpu.sync_copy(pk.at[wid, 0], ib_a)
    pltpu.sync_copy(pv.at[wid, 0], vb_a)
    pltpu.async_copy(ego.at[ib_a.at[0]], gb_a, sem_ga)
    pltpu.sync_copy(pk.at[wid, 1], ib_b)
    pltpu.sync_copy(pv.at[wid, 1], vb_b)
    pltpu.async_copy(ego.at[ib_b.at[0]], gb_b, sem_gb)

    segment(0, ib_a, vb_a, rb_a, gb_a, sb_a, sem_ga, sem_sa, False, True)
    segment(1, ib_b, vb_b, rb_b, gb_b, sb_b, sem_gb, sem_sb, False, True)

    def pipe_body(t, carry):
        segment(2 * t, ib_a, vb_a, rb_a, gb_a, sb_a, sem_ga, sem_sa, True, True)
        segment(2 * t + 1, ib_b, vb_b, rb_b, gb_b, sb_b, sem_gb, sem_sb, True, True)
        return carry

    lax.fori_loop(1, NCH // 2 - 1, pipe_body, 0)

    segment(NCH - 2, ib_a, vb_a, rb_a, gb_a, sb_a, sem_ga, sem_sa, True, False)
    segment(NCH - 1, ib_b, vb_b, rb_b, gb_b, sb_b, sem_gb, sem_sb, True, False)

    pltpu.make_async_copy(sb_a, acc.at[rb_a], sem_sa).wait()
    pltpu.make_async_copy(sb_b, acc.at[rb_b], sem_sb).wait()

    plsc.subcore_barrier()

    @pl.when(c == 0)
    def _():
        pltpu.sync_copy(acc.at[pl.ds(row0, STRIPE)],
                        part0.at[pl.ds(row0, STRIPE)])

        @pl.when(s == 0)
        def _():
            pltpu.sync_copy(acc.at[pl.ds(TAIL_OFF, TAIL0)],
                            part0.at[pl.ds(TAIL_OFF, TAIL0)])

    @pl.when(c == 1)
    def _():
        pltpu.sync_copy(acc.at[pl.ds(row0, STRIPE)],
                        part1.at[pl.ds(row0, STRIPE)])

        @pl.when(s == 0)
        def _():
            pltpu.sync_copy(acc.at[pl.ds(TAIL_OFF, TAIL0)],
                            part1.at[pl.ds(TAIL_OFF, TAIL0)])


_sc_layer = functools.partial(
    pl.kernel,
    out_type=(
        jax.ShapeDtypeStruct((N, EMB), jnp.float32),
        jax.ShapeDtypeStruct((N, EMB), jnp.float32),
    ),
    mesh=_mesh,
    scratch_types=[
        pltpu.VMEM((2, CHUNK), jnp.int32),
        pltpu.VMEM((2, CHUNK), jnp.int32),
        pltpu.VMEM((2, 16), jnp.int32),
        pltpu.VMEM((1, CHUNK), jnp.float32),
        pltpu.VMEM((1, CHUNK), jnp.float32),
        pltpu.VMEM((1, 16), jnp.float32),
        pltpu.VMEM((CHUNK,), jnp.int32),
        pltpu.VMEM((CHUNK,), jnp.int32),
        pltpu.VMEM((16,), jnp.int32),
        pltpu.VMEM((CHUNK, EMB), jnp.float32),
        pltpu.VMEM((CHUNK, EMB), jnp.float32),
        pltpu.VMEM((CHUNK, EMB), jnp.float32),
        pltpu.VMEM((CHUNK, EMB), jnp.float32),
        pltpu.VMEM_SHARED((N, EMB), jnp.float32),
        pltpu.SemaphoreType.DMA,
        pltpu.SemaphoreType.DMA,
        pltpu.SemaphoreType.DMA,
        pltpu.SemaphoreType.DMA,
    ],
)(_sc_layer_body)


_BLK = 1000


def _add2_body(a_ref, b_ref, o_ref):
    o_ref[...] = a_ref[...] + b_ref[...]


def _combine(a, b):
    return pl.pallas_call(
        _add2_body,
        grid=(N // _BLK,),
        in_specs=[pl.BlockSpec((_BLK, EMB), lambda i: (i, 0))] * 2,
        out_specs=pl.BlockSpec((_BLK, EMB), lambda i: (i, 0)),
        out_shape=jax.ShapeDtypeStruct((N, EMB), jnp.float32),
    )(a, b)


def _mean_body(e1_ref, e2_ref, p0_ref, p1_ref, o_ref):
    o_ref[...] = (e1_ref[...] + e2_ref[...] + p0_ref[...] + p1_ref[...]) * (
        1.0 / NLAYERS
    )


def _final_mean(e1, e2, p0, p1):
    return pl.pallas_call(
        _mean_body,
        grid=(N // _BLK,),
        in_specs=[pl.BlockSpec((_BLK, EMB), lambda i: (i, 0))] * 4,
        out_specs=pl.BlockSpec((_BLK, EMB), lambda i: (i, 0)),
        out_shape=jax.ShapeDtypeStruct((N, EMB), jnp.float32),
    )(e1, e2, p0, p1)


def kernel(user_emb, item_emb, adj_values, adj_indices):
    ego = jnp.concatenate([user_emb, item_emb], axis=0)
    rows = adj_indices[0].reshape(NW, EPT)
    cols = adj_indices[1].reshape(NW, EPT)
    vals = adj_values.reshape(NW, EPT)

    main = NCH * CHUNK
    pk = jnp.stack(
        [cols[:, :main].reshape(NW, NCH, CHUNK),
         rows[:, :main].reshape(NW, NCH, CHUNK)], axis=2)
    pv = vals[:, :main].reshape(NW, NCH, 1, CHUNK)
    pkt = jnp.stack([cols[:, main:], rows[:, main:]], axis=1)
    pvt = vals[:, main:].reshape(NW, 1, TAIL)
    zeros = jnp.zeros((N, EMB), jnp.float32)

    p0, p1 = _sc_layer(ego, pk, pv, pkt, pvt, zeros)
    e1 = _combine(p0, p1)
    p0, p1 = _sc_layer(e1, pk, pv, pkt, pvt, zeros)
    e2 = _combine(p0, p1)
    p0, p1 = _sc_layer(e2, pk, pv, pkt, pvt, zeros)
    out = _final_mean(e1, e2, p0, p1)
    return out[:USER_N], out[USER_N:]


# speedup vs baseline: 1.7222x; 1.7222x over previous
import functools

import jax
import jax.numpy as jnp
from jax import lax
from jax.experimental import pallas as pl
from jax.experimental.pallas import tpu as pltpu
from jax.experimental.pallas import tpu_sc as plsc

USER_N = 5000
ITEM_N = 5000
N = USER_N + ITEM_N
NNZ = 320000
EMB = 128
NLAYERS = 3

NC = 2
NS = 16
NW = NC * NS
EPT = NNZ // NW
CHUNK = 48
NCH = 208
TAIL = EPT - NCH * CHUNK
NGRP = CHUNK // 16

QSCALE = float(2.0 ** 30)
QINV = float(2.0 ** -30)

STRIPE = 624
TAIL0 = N - NS * STRIPE
TAIL_OFF = NS * STRIPE

_mesh = plsc.VectorSubcoreMesh(
    core_axis_name="c", subcore_axis_name="s", num_cores=NC, num_subcores=NS
)

_DNUMS = lax.GatherDimensionNumbers(
    offset_dims=(), collapsed_slice_dims=(0,), start_index_map=(0,))


def _splat(vals16, lane):
    return lax.gather(vals16, jnp.full((16, 1), lane, jnp.int32), _DNUMS,
                      slice_sizes=(1,),
                      mode=lax.GatherScatterMode.PROMISE_IN_BOUNDS)


def _sc_layer_body(ego, pk, pkt, zeros, part0, part1,
                   ib_a0, ib_a1, ib_b0, ib_b1, ibt, rb_a, rb_b, rbt,
                   gb_a, gb_b, sb_a, sb_b, acc,
                   sem_ga, sem_gb, sem_sa, sem_sb, sem_ia, sem_ib):
    c = lax.axis_index("c")
    s = lax.axis_index("s")
    wid = c * NS + s

    row0 = s * STRIPE
    pltpu.sync_copy(zeros.at[pl.ds(row0, STRIPE)], acc.at[pl.ds(row0, STRIPE)])

    @pl.when(s == 0)
    def _():
        pltpu.sync_copy(zeros.at[pl.ds(TAIL_OFF, TAIL0)],
                        acc.at[pl.ds(TAIL_OFF, TAIL0)])

    plsc.subcore_barrier()

    pltpu.sync_copy(pkt.at[wid], ibt)
    pltpu.async_copy(ego.at[ibt.at[0]], gb_a.at[pl.ds(0, 16)], sem_ga).wait()
    rbt[pl.ds(0, 16)] = ibt[1, pl.ds(0, 16)]
    vals16 = ibt[2, pl.ds(0, 16)].astype(jnp.float32) * QINV
    for lane in range(16):
        v = _splat(vals16, lane)
        for k in range(EMB // 16):
            sb_a[lane, pl.ds(k * 16, 16)] = gb_a[lane, pl.ds(k * 16, 16)] * v
    pltpu.sync_copy(sb_a.at[pl.ds(0, 16)], acc.at[rbt], add=True)

    def scale(ib, rb, gb, sb):
        def group(g, carry):
            rb[pl.ds(g * 16, 16)] = ib[1, pl.ds(g * 16, 16)]
            vals = ib[2, pl.ds(g * 16, 16)].astype(jnp.float32) * QINV
            for lane in range(16):
                v = _splat(vals, lane)
                e = g * 16 + lane
                for k in range(EMB // 16):
                    sb[e, pl.ds(k * 16, 16)] = gb[e, pl.ds(k * 16, 16)] * v
            return carry
        lax.fori_loop(0, NGRP, group, 0)

    def segment(j, ib_cur, ib_nxt, rb, gb, sb, sem_g, sem_s, sem_i,
                drain_s, wait_i, fire_g, fire_i):
        pltpu.make_async_copy(ego.at[ib_cur.at[0]], gb, sem_g).wait()
        if drain_s:
            pltpu.make_async_copy(sb, acc.at[rb], sem_s).wait()
        scale(ib_cur, rb, gb, sb)
        pltpu.async_copy(sb, acc.at[rb], sem_s, add=True)
        if fire_g:
            if wait_i:
                pltpu.make_async_copy(pk.at[wid, j + 2], ib_nxt, sem_i).wait()
            pltpu.async_copy(ego.at[ib_nxt.at[0]], gb, sem_g)
        if fire_i:
            pltpu.async_copy(pk.at[wid, j + 4], ib_cur, sem_i)

    pltpu.sync_copy(pk.at[wid, 0], ib_a0)
    pltpu.sync_copy(pk.at[wid, 1], ib_b0)
    pltpu.sync_copy(pk.at[wid, 2], ib_a1)
    pltpu.sync_copy(pk.at[wid, 3], ib_b1)
    pltpu.async_copy(ego.at[ib_a0.at[0]], gb_a, sem_ga)
    pltpu.async_copy(ego.at[ib_b0.at[0]], gb_b, sem_gb)

    segment(0, ib_a0, ib_a1, rb_a, gb_a, sb_a, sem_ga, sem_sa, sem_ia,
            False, False, True, True)
    segment(1, ib_b0, ib_b1, rb_b, gb_b, sb_b, sem_gb, sem_sb, sem_ib,
            False, False, True, True)
    segment(2, ib_a1, ib_a0, rb_a, gb_a, sb_a, sem_ga, sem_sa, sem_ia,
            True, True, True, True)
    segment(3, ib_b1, ib_b0, rb_b, gb_b, sb_b, sem_gb, sem_sb, sem_ib,
            True, True, True, True)

    def pipe_body(u, carry):
        j = 4 * u
        segment(j, ib_a0, ib_a1, rb_a, gb_a, sb_a, sem_ga, sem_sa, sem_ia,
                True, True, True, True)
        segment(j + 1, ib_b0, ib_b1, rb_b, gb_b, sb_b, sem_gb, sem_sb, sem_ib,
                True, True, True, True)
        segment(j + 2, ib_a1, ib_a0, rb_a, gb_a, sb_a, sem_ga, sem_sa, sem_ia,
                True, True, True, True)
        segment(j + 3, ib_b1, ib_b0, rb_b, gb_b, sb_b, sem_gb, sem_sb, sem_ib,
                True, True, True, True)
        return carry

    lax.fori_loop(1, NCH // 4 - 1, pipe_body, 0)

    segment(NCH - 4, ib_a0, ib_a1, rb_a, gb_a, sb_a, sem_ga, sem_sa, sem_ia,
            True, True, True, False)
    segment(NCH - 3, ib_b0, ib_b1, rb_b, gb_b, sb_b, sem_gb, sem_sb, sem_ib,
            True, True, True, False)
    segment(NCH - 2, ib_a1, ib_a0, rb_a, gb_a, sb_a, sem_ga, sem_sa, sem_ia,
            True, False, False, False)
    segment(NCH - 1, ib_b1, ib_b0, rb_b, gb_b, sb_b, sem_gb, sem_sb, sem_ib,
            True, False, False, False)

    pltpu.make_async_copy(sb_a, acc.at[rb_a], sem_sa).wait()
    pltpu.make_async_copy(sb_b, acc.at[rb_b], sem_sb).wait()

    plsc.subcore_barrier()

    @pl.when(c == 0)
    def _():
        pltpu.sync_copy(acc.at[pl.ds(row0, STRIPE)],
                        part0.at[pl.ds(row0, STRIPE)])

        @pl.when(s == 0)
        def _():
            pltpu.sync_copy(acc.at[pl.ds(TAIL_OFF, TAIL0)],
                            part0.at[pl.ds(TAIL_OFF, TAIL0)])

    @pl.when(c == 1)
    def _():
        pltpu.sync_copy(acc.at[pl.ds(row0, STRIPE)],
                        part1.at[pl.ds(row0, STRIPE)])

        @pl.when(s == 0)
        def _():
            pltpu.sync_copy(acc.at[pl.ds(TAIL_OFF, TAIL0)],
                            part1.at[pl.ds(TAIL_OFF, TAIL0)])


_sc_layer = functools.partial(
    pl.kernel,
    out_type=(
        jax.ShapeDtypeStruct((N, EMB), jnp.float32),
        jax.ShapeDtypeStruct((N, EMB), jnp.float32),
    ),
    mesh=_mesh,
    scratch_types=[
        pltpu.VMEM((3, CHUNK), jnp.int32),
        pltpu.VMEM((3, CHUNK), jnp.int32),
        pltpu.VMEM((3, CHUNK), jnp.int32),
        pltpu.VMEM((3, CHUNK), jnp.int32),
        pltpu.VMEM((3, TAIL), jnp.int32),
        pltpu.VMEM((CHUNK,), jnp.int32),
        pltpu.VMEM((CHUNK,), jnp.int32),
        pltpu.VMEM((TAIL,), jnp.int32),
        pltpu.VMEM((CHUNK, EMB), jnp.float32),
        pltpu.VMEM((CHUNK, EMB), jnp.float32),
        pltpu.VMEM((CHUNK, EMB), jnp.float32),
        pltpu.VMEM((CHUNK, EMB), jnp.float32),
        pltpu.VMEM_SHARED((N, EMB), jnp.float32),
        pltpu.SemaphoreType.DMA,
        pltpu.SemaphoreType.DMA,
        pltpu.SemaphoreType.DMA,
        pltpu.SemaphoreType.DMA,
        pltpu.SemaphoreType.DMA,
        pltpu.SemaphoreType.DMA,
    ],
)(_sc_layer_body)


_BLK = 1000


def _add2_body(a_ref, b_ref, o_ref):
    o_ref[...] = a_ref[...] + b_ref[...]


def _combine(a, b):
    return pl.pallas_call(
        _add2_body,
        grid=(N // _BLK,),
        in_specs=[pl.BlockSpec((_BLK, EMB), lambda i: (i, 0))] * 2,
        out_specs=pl.BlockSpec((_BLK, EMB), lambda i: (i, 0)),
        out_shape=jax.ShapeDtypeStruct((N, EMB), jnp.float32),
    )(a, b)


def _mean_body(e1_ref, e2_ref, p0_ref, p1_ref, o_ref):
    o_ref[...] = (e1_ref[...] + e2_ref[...] + p0_ref[...] + p1_ref[...]) * (
        1.0 / NLAYERS
    )


def _final_mean(e1, e2, p0, p1):
    return pl.pallas_call(
        _mean_body,
        grid=(N // _BLK,),
        in_specs=[pl.BlockSpec((_BLK, EMB), lambda i: (i, 0))] * 4,
        out_specs=pl.BlockSpec((_BLK, EMB), lambda i: (i, 0)),
        out_shape=jax.ShapeDtypeStruct((N, EMB), jnp.float32),
    )(e1, e2, p0, p1)


def kernel(user_emb, item_emb, adj_values, adj_indices):
    ego = jnp.concatenate([user_emb, item_emb], axis=0)
    rows = adj_indices[0].reshape(NW, EPT)
    cols = adj_indices[1].reshape(NW, EPT)
    qvals = jnp.round(adj_values * QSCALE).astype(jnp.int32).reshape(NW, EPT)

    main = NCH * CHUNK
    pk = jnp.stack(
        [cols[:, :main].reshape(NW, NCH, CHUNK),
         rows[:, :main].reshape(NW, NCH, CHUNK),
         qvals[:, :main].reshape(NW, NCH, CHUNK)], axis=2)
    pkt = jnp.stack([cols[:, main:], rows[:, main:], qvals[:, main:]],
                    axis=1)
    zeros = jnp.zeros((N, EMB), jnp.float32)

    p0, p1 = _sc_layer(ego, pk, pkt, zeros)
    e1 = _combine(p0, p1)
    p0, p1 = _sc_layer(e1, pk, pkt, zeros)
    e2 = _combine(p0, p1)
    p0, p1 = _sc_layer(e2, pk, pkt, zeros)
    out = _final_mean(e1, e2, p0, p1)
    return out[:USER_N], out[USER_N:]


# speedup vs baseline: 1.8521x; 1.0754x over previous
import functools

import jax
import jax.numpy as jnp
from jax import lax
from jax.experimental import pallas as pl
from jax.experimental.pallas import tpu as pltpu
from jax.experimental.pallas import tpu_sc as plsc

USER_N = 5000
ITEM_N = 5000
N = USER_N + ITEM_N
NNZ = 320000
EMB = 128
NLAYERS = 3

NC = 2
NS = 16
NW = NC * NS
EPT = NNZ // NW
CHUNK = 48
NCH = 208
TAIL = EPT - NCH * CHUNK
NGRP = CHUNK // 16

QSCALE = float(2.0 ** 30)
QINV = float(2.0 ** -30)

STRIPE = 624
TAIL0 = N - NS * STRIPE
TAIL_OFF = NS * STRIPE

_mesh = plsc.VectorSubcoreMesh(
    core_axis_name="c", subcore_axis_name="s", num_cores=NC, num_subcores=NS
)

_DNUMS = lax.GatherDimensionNumbers(
    offset_dims=(), collapsed_slice_dims=(0,), start_index_map=(0,))


def _splat(vals16, lane):
    return lax.gather(vals16, jnp.full((16, 1), lane, jnp.int32), _DNUMS,
                      slice_sizes=(1,),
                      mode=lax.GatherScatterMode.PROMISE_IN_BOUNDS)


def _sc_layer_body(ego, pk, pkt, zeros, part0, part1,
                   ib0, ib1, ib2, ib3, ibt, rb0, rb1, rb2, rb3, rbt,
                   gb0, gb1, gb2, gb3, acc,
                   sg0, sg1, sg2, sg3, ss0, ss1, ss2, ss3,
                   si0, si1, si2, si3):
    ib = (ib0, ib1, ib2, ib3)
    rb = (rb0, rb1, rb2, rb3)
    gb = (gb0, gb1, gb2, gb3)
    sem_g = (sg0, sg1, sg2, sg3)
    sem_s = (ss0, ss1, ss2, ss3)
    sem_i = (si0, si1, si2, si3)
    c = lax.axis_index("c")
    s = lax.axis_index("s")
    wid = c * NS + s

    row0 = s * STRIPE
    pltpu.sync_copy(zeros.at[pl.ds(row0, STRIPE)], acc.at[pl.ds(row0, STRIPE)])

    @pl.when(s == 0)
    def _():
        pltpu.sync_copy(zeros.at[pl.ds(TAIL_OFF, TAIL0)],
                        acc.at[pl.ds(TAIL_OFF, TAIL0)])

    plsc.subcore_barrier()

    pltpu.sync_copy(pkt.at[wid], ibt)
    pltpu.async_copy(ego.at[ibt.at[0]], gb0.at[pl.ds(0, 16)], sg0).wait()
    rbt[pl.ds(0, 16)] = ibt[1, pl.ds(0, 16)]
    vals16 = ibt[2, pl.ds(0, 16)].astype(jnp.float32) * QINV
    for lane in range(16):
        v = _splat(vals16, lane)
        for k in range(EMB // 16):
            gb0[lane, pl.ds(k * 16, 16)] = gb0[lane, pl.ds(k * 16, 16)] * v
    pltpu.sync_copy(gb0.at[pl.ds(0, 16)], acc.at[rbt], add=True)

    def scale(ib, rb, gb):
        def group(g, carry):
            rb[pl.ds(g * 16, 16)] = ib[1, pl.ds(g * 16, 16)]
            vals = ib[2, pl.ds(g * 16, 16)].astype(jnp.float32) * QINV
            for lane in range(16):
                v = _splat(vals, lane)
                e = g * 16 + lane
                for k in range(EMB // 16):
                    gb[e, pl.ds(k * 16, 16)] = gb[e, pl.ds(k * 16, 16)] * v
            return carry
        lax.fori_loop(0, NGRP, group, 0)

    def segment(j, r, drain_s, wait_i, fire_g, fire_i):
        rn = (r + 2) % 4
        pltpu.make_async_copy(ego.at[ib[r].at[0]], gb[r], sem_g[r]).wait()
        if drain_s:
            pltpu.make_async_copy(gb[rn], acc.at[rb[rn]], sem_s[rn]).wait()
        if fire_g:
            if wait_i:
                pltpu.make_async_copy(pk.at[wid, j + 2], ib[rn], sem_i[rn]).wait()
            pltpu.async_copy(ego.at[ib[rn].at[0]], gb[rn], sem_g[rn])
        scale(ib[r], rb[r], gb[r])
        pltpu.async_copy(gb[r], acc.at[rb[r]], sem_s[r], add=True)
        if fire_i:
            pltpu.async_copy(pk.at[wid, j + 4], ib[r], sem_i[r])

    pltpu.sync_copy(pk.at[wid, 0], ib[0])
    pltpu.sync_copy(pk.at[wid, 1], ib[1])
    pltpu.sync_copy(pk.at[wid, 2], ib[2])
    pltpu.sync_copy(pk.at[wid, 3], ib[3])
    pltpu.async_copy(ego.at[ib[0].at[0]], gb[0], sem_g[0])
    pltpu.async_copy(ego.at[ib[1].at[0]], gb[1], sem_g[1])

    segment(0, 0, False, False, True, True)
    segment(1, 1, False, False, True, True)
    segment(2, 2, True, True, True, True)
    segment(3, 3, True, True, True, True)

    def pipe_body(u, carry):
        j = 4 * u
        segment(j, 0, True, True, True, True)
        segment(j + 1, 1, True, True, True, True)
        segment(j + 2, 2, True, True, True, True)
        segment(j + 3, 3, True, True, True, True)
        return carry

    lax.fori_loop(1, NCH // 4 - 1, pipe_body, 0)

    segment(NCH - 4, 0, True, True, True, False)
    segment(NCH - 3, 1, True, True, True, False)
    segment(NCH - 2, 2, True, False, False, False)
    segment(NCH - 1, 3, True, False, False, False)

    for r in (2, 3):
        pltpu.make_async_copy(gb[r], acc.at[rb[r]], sem_s[r]).wait()

    plsc.subcore_barrier()

    @pl.when(c == 0)
    def _():
        pltpu.sync_copy(acc.at[pl.ds(row0, STRIPE)],
                        part0.at[pl.ds(row0, STRIPE)])

        @pl.when(s == 0)
        def _():
            pltpu.sync_copy(acc.at[pl.ds(TAIL_OFF, TAIL0)],
                            part0.at[pl.ds(TAIL_OFF, TAIL0)])

    @pl.when(c == 1)
    def _():
        pltpu.sync_copy(acc.at[pl.ds(row0, STRIPE)],
                        part1.at[pl.ds(row0, STRIPE)])

        @pl.when(s == 0)
        def _():
            pltpu.sync_copy(acc.at[pl.ds(TAIL_OFF, TAIL0)],
                            part1.at[pl.ds(TAIL_OFF, TAIL0)])


_sc_layer = functools.partial(
    pl.kernel,
    out_type=(
        jax.ShapeDtypeStruct((N, EMB), jnp.float32),
        jax.ShapeDtypeStruct((N, EMB), jnp.float32),
    ),
    mesh=_mesh,
    scratch_types=[
        pltpu.VMEM((3, CHUNK), jnp.int32),
        pltpu.VMEM((3, CHUNK), jnp.int32),
        pltpu.VMEM((3, CHUNK), jnp.int32),
        pltpu.VMEM((3, CHUNK), jnp.int32),
        pltpu.VMEM((3, TAIL), jnp.int32),
        pltpu.VMEM((CHUNK,), jnp.int32),
        pltpu.VMEM((CHUNK,), jnp.int32),
        pltpu.VMEM((CHUNK,), jnp.int32),
        pltpu.VMEM((CHUNK,), jnp.int32),
        pltpu.VMEM((TAIL,), jnp.int32),
        pltpu.VMEM((CHUNK, EMB), jnp.float32),
        pltpu.VMEM((CHUNK, EMB), jnp.float32),
        pltpu.VMEM((CHUNK, EMB), jnp.float32),
        pltpu.VMEM((CHUNK, EMB), jnp.float32),
        pltpu.VMEM_SHARED((N, EMB), jnp.float32),
    ] + [pltpu.SemaphoreType.DMA] * 12,
)(_sc_layer_body)


_BLK = 1000


def _add2_body(a_ref, b_ref, o_ref):
    o_ref[...] = a_ref[...] + b_ref[...]


def _combine(a, b):
    return pl.pallas_call(
        _add2_body,
        grid=(N // _BLK,),
        in_specs=[pl.BlockSpec((_BLK, EMB), lambda i: (i, 0))] * 2,
        out_specs=pl.BlockSpec((_BLK, EMB), lambda i: (i, 0)),
        out_shape=jax.ShapeDtypeStruct((N, EMB), jnp.float32),
    )(a, b)


def _mean_body(e1_ref, e2_ref, p0_ref, p1_ref, o_ref):
    o_ref[...] = (e1_ref[...] + e2_ref[...] + p0_ref[...] + p1_ref[...]) * (
        1.0 / NLAYERS
    )


def _final_mean(e1, e2, p0, p1):
    return pl.pallas_call(
        _mean_body,
        grid=(N // _BLK,),
        in_specs=[pl.BlockSpec((_BLK, EMB), lambda i: (i, 0))] * 4,
        out_specs=pl.BlockSpec((_BLK, EMB), lambda i: (i, 0)),
        out_shape=jax.ShapeDtypeStruct((N, EMB), jnp.float32),
    )(e1, e2, p0, p1)


def kernel(user_emb, item_emb, adj_values, adj_indices):
    ego = jnp.concatenate([user_emb, item_emb], axis=0)
    rows = adj_indices[0].reshape(NW, EPT)
    cols = adj_indices[1].reshape(NW, EPT)
    qvals = jnp.round(adj_values * QSCALE).astype(jnp.int32).reshape(NW, EPT)

    main = NCH * CHUNK
    pk = jnp.stack(
        [cols[:, :main].reshape(NW, NCH, CHUNK),
         rows[:, :main].reshape(NW, NCH, CHUNK),
         qvals[:, :main].reshape(NW, NCH, CHUNK)], axis=2)
    pkt = jnp.stack([cols[:, main:], rows[:, main:], qvals[:, main:]],
                    axis=1)
    zeros = jnp.zeros((N, EMB), jnp.float32)

    p0, p1 = _sc_layer(ego, pk, pkt, zeros)
    e1 = _combine(p0, p1)
    p0, p1 = _sc_layer(e1, pk, pkt, zeros)
    e2 = _combine(p0, p1)
    p0, p1 = _sc_layer(e2, pk, pkt, zeros)
    out = _final_mean(e1, e2, p0, p1)
    return out[:USER_N], out[USER_N:]
